# Initial kernel scaffold; baseline (speedup 1.0000x reference)
import jax
import jax.numpy as jnp
from jax.experimental import pallas as pl


def kernel(x, edge_index, batch, W1, b1, W2, b2, Wf1, bf1, Wf2, bf2):
    raise NotImplementedError("write your pallas kernel here")


# speedup vs baseline: 5.3758x; 5.3758x over previous
import functools

import jax
import jax.numpy as jnp
from jax import lax
from jax.experimental import pallas as pl
from jax.experimental.pallas import tpu as pltpu
from jax.experimental.pallas import tpu_sc as plsc

N_NODES = 10000
HID = 64
N_GRAPHS = 64
N_CLASSES = 10

NUM_SC = 2
NUM_TILES = 16
CHUNK = 128
CH_PER_TILE = 80
E_PER_TILE = CHUNK * CH_PER_TILE
E_PAD = NUM_SC * NUM_TILES * E_PER_TILE
ACC_ROWS = 10240
ZROWS = ACC_ROWS // NUM_TILES


def _mp_body(h_hbm, src_hbm, dst_hbm, z_hbm, out_hbm,
             src_v, dst_v, rows_v, zero_v, accum, sem):
    c = lax.axis_index("c")
    s = lax.axis_index("s")
    g = c * NUM_TILES + s

    pltpu.sync_copy(z_hbm, zero_v)
    pltpu.sync_copy(zero_v, accum.at[pl.ds(s * ZROWS, ZROWS)])
    pltpu.sync_copy(src_hbm.at[g], src_v)
    pltpu.sync_copy(dst_hbm.at[g], dst_v)
    plsc.subcore_barrier()

    def chunk(j, carry):
        pltpu.async_copy(h_hbm.at[src_v.at[j]], rows_v, sem).wait()
        pltpu.sync_copy(rows_v, accum.at[dst_v.at[j]], add=True)
        return carry

    lax.fori_loop(0, CH_PER_TILE, chunk, 0)
    plsc.subcore_barrier()
    pltpu.sync_copy(accum.at[pl.ds(s * ZROWS, ZROWS)],
                    out_hbm.at[c, pl.ds(s * ZROWS, ZROWS)])


@functools.cache
def _mp_call_builder():
    return functools.partial(
        pl.kernel,
        out_type=jax.ShapeDtypeStruct((NUM_SC, ACC_ROWS, HID), jnp.float32),
        mesh=plsc.VectorSubcoreMesh(core_axis_name="c", subcore_axis_name="s"),
        compiler_params=pltpu.CompilerParams(use_tc_tiling_on_sc=False),
        scratch_types=[
            pltpu.VMEM((CH_PER_TILE, CHUNK), jnp.int32),
            pltpu.VMEM((CH_PER_TILE, CHUNK), jnp.int32),
            pltpu.VMEM((CHUNK, HID), jnp.float32),
            pltpu.VMEM((ZROWS, HID), jnp.float32),
            pltpu.VMEM_SHARED((ACC_ROWS, HID), jnp.float32),
            pltpu.SemaphoreType.DMA,
        ],
    )(_mp_body)


def _mp_call(h, src_p, dst_p, zblk):
    return _mp_call_builder()(h, src_p, dst_p, zblk)


def _mm1_body(x_ref, w_ref, o_ref):
    o_ref[:] = jnp.dot(x_ref[:], w_ref[:], preferred_element_type=jnp.float32)


def _mm2_body(p0_ref, p1_ref, b_ref, w_ref, o_ref):
    h = jnp.maximum(p0_ref[0] + p1_ref[0] + b_ref[:], 0.0)
    o_ref[:] = jnp.dot(h, w_ref[:], preferred_element_type=jnp.float32)


def _head_body(q0_ref, q1_ref, b2_ref, batch_ref, wf1_ref, bf1_ref,
               wf2_ref, bf2_ref, o_ref):
    h = jnp.maximum(q0_ref[0] + q1_ref[0] + b2_ref[:], 0.0)
    gid = lax.broadcasted_iota(jnp.int32, (N_NODES, N_GRAPHS), 1)
    oh = jnp.where(batch_ref[:] == gid, 1.0, 0.0)
    sums = lax.dot_general(oh, h, (((0,), (0,)), ((), ())),
                           preferred_element_type=jnp.float32)
    ones = jnp.ones((N_NODES, 8), jnp.float32)
    counts = lax.dot_general(oh, ones, (((0,), (0,)), ((), ())),
                             preferred_element_type=jnp.float32)[:, :1]
    pooled = sums / jnp.maximum(counts, 1.0)
    z = jnp.maximum(jnp.dot(pooled, wf1_ref[:],
                            preferred_element_type=jnp.float32) + bf1_ref[:], 0.0)
    logits = jnp.dot(z, wf2_ref[:],
                     preferred_element_type=jnp.float32) + bf2_ref[:]
    m = jnp.max(logits, axis=1, keepdims=True)
    lse = jnp.log(jnp.sum(jnp.exp(logits - m), axis=1, keepdims=True))
    o_ref[:] = logits - m - lse


_ROWB = 1000


def _mm1(x, W1):
    return pl.pallas_call(
        _mm1_body,
        grid=(N_NODES // _ROWB,),
        in_specs=[pl.BlockSpec((_ROWB, 128), lambda i: (i, 0)),
                  pl.BlockSpec((128, HID), lambda i: (0, 0))],
        out_specs=pl.BlockSpec((_ROWB, HID), lambda i: (i, 0)),
        out_shape=jax.ShapeDtypeStruct((N_NODES, HID), jnp.float32),
    )(x, W1)


def _mm2(p, b, W):
    return pl.pallas_call(
        _mm2_body,
        grid=(N_NODES // _ROWB,),
        in_specs=[pl.BlockSpec((1, _ROWB, HID), lambda i: (0, i, 0)),
                  pl.BlockSpec((1, _ROWB, HID), lambda i: (1, i, 0)),
                  pl.BlockSpec((1, HID), lambda i: (0, 0)),
                  pl.BlockSpec((HID, HID), lambda i: (0, 0))],
        out_specs=pl.BlockSpec((_ROWB, HID), lambda i: (i, 0)),
        out_shape=jax.ShapeDtypeStruct((N_NODES, HID), jnp.float32),
    )(p, p, b, W)


def _head(q, b2, batch2d, Wf1, bf1, Wf2, bf2):
    return pl.pallas_call(
        _head_body,
        grid=(1,),
        in_specs=[pl.BlockSpec((1, N_NODES, HID), lambda i: (0, 0, 0)),
                  pl.BlockSpec((1, N_NODES, HID), lambda i: (1, 0, 0)),
                  pl.BlockSpec((1, HID), lambda i: (0, 0)),
                  pl.BlockSpec((N_NODES, 1), lambda i: (0, 0)),
                  pl.BlockSpec((HID, 32), lambda i: (0, 0)),
                  pl.BlockSpec((1, 32), lambda i: (0, 0)),
                  pl.BlockSpec((32, N_CLASSES), lambda i: (0, 0)),
                  pl.BlockSpec((1, N_CLASSES), lambda i: (0, 0))],
        out_specs=pl.BlockSpec((N_GRAPHS, N_CLASSES), lambda i: (0, 0)),
        out_shape=jax.ShapeDtypeStruct((N_GRAPHS, N_CLASSES), jnp.float32),
    )(q, q, b2, batch2d, Wf1, bf1, Wf2, bf2)


def kernel(x, edge_index, batch, W1, b1, W2, b2, Wf1, bf1, Wf2, bf2):
    src = edge_index[0]
    dst = edge_index[1]
    pad = E_PAD - src.shape[0]
    src_p = jnp.concatenate(
        [src, jnp.zeros((pad,), jnp.int32)]).reshape(
        NUM_SC * NUM_TILES, CH_PER_TILE, CHUNK)
    dst_p = jnp.concatenate(
        [dst, jnp.full((pad,), N_NODES, jnp.int32)]).reshape(
        NUM_SC * NUM_TILES, CH_PER_TILE, CHUNK)
    zblk = jnp.zeros((ZROWS, HID), jnp.float32)

    b1r = b1.reshape(1, HID)
    b2r = b2.reshape(1, HID)
    bf1r = bf1.reshape(1, 32)
    bf2r = bf2.reshape(1, N_CLASSES)
    batch2d = batch.reshape(N_NODES, 1)

    h1 = _mm1(x, W1)
    p = _mp_call(h1, src_p, dst_p, zblk)
    h2 = _mm2(p, b1r, W2)
    q = _mp_call(h2, src_p, dst_p, zblk)
    return _head(q, b2r, batch2d, Wf1, bf1r, Wf2, bf2r)


# speedup vs baseline: 6.5612x; 1.2205x over previous
import functools

import jax
import jax.numpy as jnp
from jax import lax
from jax.experimental import pallas as pl
from jax.experimental.pallas import tpu as pltpu
from jax.experimental.pallas import tpu_sc as plsc

N_NODES = 10000
HID = 64
N_GRAPHS = 64
N_CLASSES = 10

NUM_SC = 2
NUM_TILES = 16
CHUNK = 128
CH_PER_TILE = 80
E_PER_TILE = CHUNK * CH_PER_TILE
E_PAD = NUM_SC * NUM_TILES * E_PER_TILE
NBUF = 4
ACC_ROWS = 10240
ZROWS = ACC_ROWS // NUM_TILES


def _mp_body(h_hbm, src_hbm, dst_hbm, z_hbm, out_hbm,
             src_v, dst_v, rows_v, accum, sem):
    c = lax.axis_index("c")
    s = lax.axis_index("s")
    g = c * NUM_TILES + s

    pltpu.sync_copy(z_hbm, accum.at[pl.ds(s * ZROWS, ZROWS)])
    pltpu.sync_copy(src_hbm.at[g], src_v)
    pltpu.sync_copy(dst_hbm.at[g], dst_v)
    plsc.subcore_barrier()

    for b in range(NBUF - 1):
        pltpu.async_copy(h_hbm.at[src_v.at[b]], rows_v.at[b], sem.at[b])

    def outer(i, carry):
        jj0 = i * NBUF
        for b in range(NBUF):
            jj = jj0 + b
            pltpu.make_async_copy(
                h_hbm.at[src_v.at[jj]], rows_v.at[b], sem.at[b]).wait()
            nxt = jj + NBUF - 1
            nb = (b + NBUF - 1) % NBUF

            @pl.when(nxt < CH_PER_TILE)
            def _():
                pltpu.async_copy(
                    h_hbm.at[src_v.at[nxt]], rows_v.at[nb], sem.at[nb])

            pltpu.sync_copy(rows_v.at[b], accum.at[dst_v.at[jj]], add=True)
        return carry

    lax.fori_loop(0, CH_PER_TILE // NBUF, outer, 0)
    plsc.subcore_barrier()
    pltpu.sync_copy(accum.at[pl.ds(s * ZROWS, ZROWS)],
                    out_hbm.at[c, pl.ds(s * ZROWS, ZROWS)])


@functools.cache
def _mp_call_builder():
    return functools.partial(
        pl.kernel,
        out_type=jax.ShapeDtypeStruct((NUM_SC, ACC_ROWS, HID), jnp.float32),
        mesh=plsc.VectorSubcoreMesh(core_axis_name="c", subcore_axis_name="s"),
        compiler_params=pltpu.CompilerParams(use_tc_tiling_on_sc=False),
        scratch_types=[
            pltpu.VMEM((CH_PER_TILE, CHUNK), jnp.int32),
            pltpu.VMEM((CH_PER_TILE, CHUNK), jnp.int32),
            pltpu.VMEM((NBUF, CHUNK, HID), jnp.float32),
            pltpu.VMEM_SHARED((ACC_ROWS, HID), jnp.float32),
            pltpu.SemaphoreType.DMA((NBUF,)),
        ],
    )(_mp_body)


def _mp_call(h, src_p, dst_p, zblk):
    return _mp_call_builder()(h, src_p, dst_p, zblk)


def _mm1_body(x_ref, w_ref, o_ref):
    o_ref[:] = jnp.dot(x_ref[:], w_ref[:], preferred_element_type=jnp.float32)


def _mm2_body(p0_ref, p1_ref, b_ref, w_ref, o_ref):
    h = jnp.maximum(p0_ref[0] + p1_ref[0] + b_ref[:], 0.0)
    o_ref[:] = jnp.dot(h, w_ref[:], preferred_element_type=jnp.float32)


def _head_body(q0_ref, q1_ref, b2_ref, batch_ref, wf1_ref, bf1_ref,
               wf2_ref, bf2_ref, o_ref):
    h = jnp.maximum(q0_ref[0] + q1_ref[0] + b2_ref[:], 0.0)
    gid = lax.broadcasted_iota(jnp.int32, (N_NODES, N_GRAPHS), 1)
    oh = jnp.where(batch_ref[:] == gid, 1.0, 0.0)
    sums = lax.dot_general(oh, h, (((0,), (0,)), ((), ())),
                           preferred_element_type=jnp.float32)
    ones = jnp.ones((N_NODES, 8), jnp.float32)
    counts = lax.dot_general(oh, ones, (((0,), (0,)), ((), ())),
                             preferred_element_type=jnp.float32)[:, :1]
    pooled = sums / jnp.maximum(counts, 1.0)
    z = jnp.maximum(jnp.dot(pooled, wf1_ref[:],
                            preferred_element_type=jnp.float32) + bf1_ref[:], 0.0)
    logits = jnp.dot(z, wf2_ref[:],
                     preferred_element_type=jnp.float32) + bf2_ref[:]
    m = jnp.max(logits, axis=1, keepdims=True)
    lse = jnp.log(jnp.sum(jnp.exp(logits - m), axis=1, keepdims=True))
    o_ref[:] = logits - m - lse


_ROWB = 1000


def _mm1(x, W1):
    return pl.pallas_call(
        _mm1_body,
        grid=(N_NODES // _ROWB,),
        in_specs=[pl.BlockSpec((_ROWB, 128), lambda i: (i, 0)),
                  pl.BlockSpec((128, HID), lambda i: (0, 0))],
        out_specs=pl.BlockSpec((_ROWB, HID), lambda i: (i, 0)),
        out_shape=jax.ShapeDtypeStruct((N_NODES, HID), jnp.float32),
    )(x, W1)


def _mm2(p, b, W):
    return pl.pallas_call(
        _mm2_body,
        grid=(N_NODES // _ROWB,),
        in_specs=[pl.BlockSpec((1, _ROWB, HID), lambda i: (0, i, 0)),
                  pl.BlockSpec((1, _ROWB, HID), lambda i: (1, i, 0)),
                  pl.BlockSpec((1, HID), lambda i: (0, 0)),
                  pl.BlockSpec((HID, HID), lambda i: (0, 0))],
        out_specs=pl.BlockSpec((_ROWB, HID), lambda i: (i, 0)),
        out_shape=jax.ShapeDtypeStruct((N_NODES, HID), jnp.float32),
    )(p, p, b, W)


def _head(q, b2, batch2d, Wf1, bf1, Wf2, bf2):
    return pl.pallas_call(
        _head_body,
        grid=(1,),
        in_specs=[pl.BlockSpec((1, N_NODES, HID), lambda i: (0, 0, 0)),
                  pl.BlockSpec((1, N_NODES, HID), lambda i: (1, 0, 0)),
                  pl.BlockSpec((1, HID), lambda i: (0, 0)),
                  pl.BlockSpec((N_NODES, 1), lambda i: (0, 0)),
                  pl.BlockSpec((HID, 32), lambda i: (0, 0)),
                  pl.BlockSpec((1, 32), lambda i: (0, 0)),
                  pl.BlockSpec((32, N_CLASSES), lambda i: (0, 0)),
                  pl.BlockSpec((1, N_CLASSES), lambda i: (0, 0))],
        out_specs=pl.BlockSpec((N_GRAPHS, N_CLASSES), lambda i: (0, 0)),
        out_shape=jax.ShapeDtypeStruct((N_GRAPHS, N_CLASSES), jnp.float32),
    )(q, q, b2, batch2d, Wf1, bf1, Wf2, bf2)


def kernel(x, edge_index, batch, W1, b1, W2, b2, Wf1, bf1, Wf2, bf2):
    src = edge_index[0]
    dst = edge_index[1]
    pad = E_PAD - src.shape[0]
    src_p = jnp.concatenate(
        [src, jnp.zeros((pad,), jnp.int32)]).reshape(
        NUM_SC * NUM_TILES, CH_PER_TILE, CHUNK)
    dst_p = jnp.concatenate(
        [dst, jnp.full((pad,), N_NODES, jnp.int32)]).reshape(
        NUM_SC * NUM_TILES, CH_PER_TILE, CHUNK)
    zblk = jnp.zeros((ZROWS, HID), jnp.float32)

    b1r = b1.reshape(1, HID)
    b2r = b2.reshape(1, HID)
    bf1r = bf1.reshape(1, 32)
    bf2r = bf2.reshape(1, N_CLASSES)
    batch2d = batch.reshape(N_NODES, 1)

    h1 = _mm1(x, W1)
    p = _mp_call(h1, src_p, dst_p, zblk)
    h2 = _mm2(p, b1r, W2)
    q = _mp_call(h2, src_p, dst_p, zblk)
    return _head(q, b2r, batch2d, Wf1, bf1r, Wf2, bf2r)


# speedup vs baseline: 12.8658x; 1.9609x over previous
import functools

import jax
import jax.numpy as jnp
from jax import lax
from jax.experimental import pallas as pl
from jax.experimental.pallas import tpu as pltpu
from jax.experimental.pallas import tpu_sc as plsc

N_NODES = 10000
HID = 64
N_GRAPHS = 64
N_CLASSES = 10

NUM_SC = 2
NUM_TILES = 16
CHUNK = 128
CH_PER_TILE = 80
E_PER_TILE = CHUNK * CH_PER_TILE
E_PAD = NUM_SC * NUM_TILES * E_PER_TILE
NBUF = 2
HROWS = N_NODES // NUM_TILES
ACC_ROWS = 10240
ZROWS = ACC_ROWS // NUM_TILES


def _mp_body(h_hbm, src_hbm, dst_hbm, z_hbm, out_hbm,
             src_v, dst_v, rows_v, htab, accum, sem):
    c = lax.axis_index("c")
    s = lax.axis_index("s")
    g = c * NUM_TILES + s

    pltpu.sync_copy(z_hbm, accum.at[pl.ds(s * ZROWS, ZROWS)])
    pltpu.sync_copy(h_hbm.at[pl.ds(s * HROWS, HROWS)],
                    htab.at[pl.ds(s * HROWS, HROWS)])
    pltpu.sync_copy(src_hbm.at[g], src_v)
    pltpu.sync_copy(dst_hbm.at[g], dst_v)
    plsc.subcore_barrier()

    for b in range(NBUF - 1):
        pltpu.async_copy(htab.at[src_v.at[b]], rows_v.at[b], sem.at[b])

    def outer(i, carry):
        jj0 = i * NBUF
        for b in range(NBUF):
            jj = jj0 + b
            pltpu.make_async_copy(
                htab.at[src_v.at[jj]], rows_v.at[b], sem.at[b]).wait()
            nxt = jj + NBUF - 1
            nb = (b + NBUF - 1) % NBUF

            @pl.when(nxt < CH_PER_TILE)
            def _():
                pltpu.async_copy(
                    htab.at[src_v.at[nxt]], rows_v.at[nb], sem.at[nb])

            pltpu.sync_copy(rows_v.at[b], accum.at[dst_v.at[jj]], add=True)
        return carry

    lax.fori_loop(0, CH_PER_TILE // NBUF, outer, 0)
    plsc.subcore_barrier()
    pltpu.sync_copy(accum.at[pl.ds(s * ZROWS, ZROWS)],
                    out_hbm.at[c, pl.ds(s * ZROWS, ZROWS)])


@functools.cache
def _mp_call_builder():
    return functools.partial(
        pl.kernel,
        out_type=jax.ShapeDtypeStruct((NUM_SC, ACC_ROWS, HID), jnp.float32),
        mesh=plsc.VectorSubcoreMesh(core_axis_name="c", subcore_axis_name="s"),
        compiler_params=pltpu.CompilerParams(use_tc_tiling_on_sc=False),
        scratch_types=[
            pltpu.VMEM((CH_PER_TILE, CHUNK), jnp.int32),
            pltpu.VMEM((CH_PER_TILE, CHUNK), jnp.int32),
            pltpu.VMEM((NBUF, CHUNK, HID), jnp.float32),
            pltpu.VMEM_SHARED((ACC_ROWS, HID), jnp.float32),
            pltpu.VMEM_SHARED((ACC_ROWS, HID), jnp.float32),
            pltpu.SemaphoreType.DMA((NBUF,)),
        ],
    )(_mp_body)


def _mp_call(h, src_p, dst_p, zblk):
    return _mp_call_builder()(h, src_p, dst_p, zblk)


def _mm1_body(x_ref, w_ref, o_ref):
    o_ref[:] = jnp.dot(x_ref[:], w_ref[:], preferred_element_type=jnp.float32)


def _mm2_body(p0_ref, p1_ref, b_ref, w_ref, o_ref):
    h = jnp.maximum(p0_ref[0] + p1_ref[0] + b_ref[:], 0.0)
    o_ref[:] = jnp.dot(h, w_ref[:], preferred_element_type=jnp.float32)


def _head_body(q0_ref, q1_ref, b2_ref, batch_ref, wf1_ref, bf1_ref,
               wf2_ref, bf2_ref, o_ref):
    h = jnp.maximum(q0_ref[0] + q1_ref[0] + b2_ref[:], 0.0)
    gid = lax.broadcasted_iota(jnp.int32, (N_NODES, N_GRAPHS), 1)
    oh = jnp.where(batch_ref[:] == gid, 1.0, 0.0)
    sums = lax.dot_general(oh, h, (((0,), (0,)), ((), ())),
                           preferred_element_type=jnp.float32)
    ones = jnp.ones((N_NODES, 8), jnp.float32)
    counts = lax.dot_general(oh, ones, (((0,), (0,)), ((), ())),
                             preferred_element_type=jnp.float32)[:, :1]
    pooled = sums / jnp.maximum(counts, 1.0)
    z = jnp.maximum(jnp.dot(pooled, wf1_ref[:],
                            preferred_element_type=jnp.float32) + bf1_ref[:], 0.0)
    logits = jnp.dot(z, wf2_ref[:],
                     preferred_element_type=jnp.float32) + bf2_ref[:]
    m = jnp.max(logits, axis=1, keepdims=True)
    lse = jnp.log(jnp.sum(jnp.exp(logits - m), axis=1, keepdims=True))
    o_ref[:] = logits - m - lse


_ROWB = 1000


def _mm1(x, W1):
    return pl.pallas_call(
        _mm1_body,
        grid=(N_NODES // _ROWB,),
        in_specs=[pl.BlockSpec((_ROWB, 128), lambda i: (i, 0)),
                  pl.BlockSpec((128, HID), lambda i: (0, 0))],
        out_specs=pl.BlockSpec((_ROWB, HID), lambda i: (i, 0)),
        out_shape=jax.ShapeDtypeStruct((N_NODES, HID), jnp.float32),
    )(x, W1)


def _mm2(p, b, W):
    return pl.pallas_call(
        _mm2_body,
        grid=(N_NODES // _ROWB,),
        in_specs=[pl.BlockSpec((1, _ROWB, HID), lambda i: (0, i, 0)),
                  pl.BlockSpec((1, _ROWB, HID), lambda i: (1, i, 0)),
                  pl.BlockSpec((1, HID), lambda i: (0, 0)),
                  pl.BlockSpec((HID, HID), lambda i: (0, 0))],
        out_specs=pl.BlockSpec((_ROWB, HID), lambda i: (i, 0)),
        out_shape=jax.ShapeDtypeStruct((N_NODES, HID), jnp.float32),
    )(p, p, b, W)


def _head(q, b2, batch2d, Wf1, bf1, Wf2, bf2):
    return pl.pallas_call(
        _head_body,
        grid=(1,),
        in_specs=[pl.BlockSpec((1, N_NODES, HID), lambda i: (0, 0, 0)),
                  pl.BlockSpec((1, N_NODES, HID), lambda i: (1, 0, 0)),
                  pl.BlockSpec((1, HID), lambda i: (0, 0)),
                  pl.BlockSpec((N_NODES, 1), lambda i: (0, 0)),
                  pl.BlockSpec((HID, 32), lambda i: (0, 0)),
                  pl.BlockSpec((1, 32), lambda i: (0, 0)),
                  pl.BlockSpec((32, N_CLASSES), lambda i: (0, 0)),
                  pl.BlockSpec((1, N_CLASSES), lambda i: (0, 0))],
        out_specs=pl.BlockSpec((N_GRAPHS, N_CLASSES), lambda i: (0, 0)),
        out_shape=jax.ShapeDtypeStruct((N_GRAPHS, N_CLASSES), jnp.float32),
    )(q, q, b2, batch2d, Wf1, bf1, Wf2, bf2)


def kernel(x, edge_index, batch, W1, b1, W2, b2, Wf1, bf1, Wf2, bf2):
    src = edge_index[0]
    dst = edge_index[1]
    pad = E_PAD - src.shape[0]
    src_p = jnp.concatenate(
        [src, jnp.zeros((pad,), jnp.int32)]).reshape(
        NUM_SC * NUM_TILES, CH_PER_TILE, CHUNK)
    dst_p = jnp.concatenate(
        [dst, jnp.full((pad,), N_NODES, jnp.int32)]).reshape(
        NUM_SC * NUM_TILES, CH_PER_TILE, CHUNK)
    zblk = jnp.zeros((ZROWS, HID), jnp.float32)

    b1r = b1.reshape(1, HID)
    b2r = b2.reshape(1, HID)
    bf1r = bf1.reshape(1, 32)
    bf2r = bf2.reshape(1, N_CLASSES)
    batch2d = batch.reshape(N_NODES, 1)

    h1 = _mm1(x, W1)
    p = _mp_call(h1, src_p, dst_p, zblk)
    h2 = _mm2(p, b1r, W2)
    q = _mp_call(h2, src_p, dst_p, zblk)
    return _head(q, b2r, batch2d, Wf1, bf1r, Wf2, bf2r)


# speedup vs baseline: 13.0184x; 1.0119x over previous
import functools

import jax
import jax.numpy as jnp
from jax import lax
from jax.experimental import pallas as pl
from jax.experimental.pallas import tpu as pltpu
from jax.experimental.pallas import tpu_sc as plsc

N_NODES = 10000
HID = 64
N_GRAPHS = 64
N_CLASSES = 10

NUM_SC = 2
NUM_TILES = 16
CHUNK = 128
CH_PER_TILE = 81
E_PER_TILE = CHUNK * CH_PER_TILE
E_PAD = NUM_SC * NUM_TILES * E_PER_TILE
NBUF = 3
HROWS = N_NODES // NUM_TILES
ACC_ROWS = 10048
ZROWS = ACC_ROWS // NUM_TILES


def _mp_body(h_hbm, src_hbm, dst_hbm, z_hbm, out_hbm,
             src_v, dst_v, rows_v, htab, accum, sem):
    c = lax.axis_index("c")
    s = lax.axis_index("s")
    g = c * NUM_TILES + s

    pltpu.sync_copy(z_hbm, accum.at[pl.ds(s * ZROWS, ZROWS)])
    pltpu.sync_copy(h_hbm.at[pl.ds(s * HROWS, HROWS)],
                    htab.at[pl.ds(s * HROWS, HROWS)])
    pltpu.sync_copy(src_hbm.at[g], src_v)
    pltpu.sync_copy(dst_hbm.at[g], dst_v)
    plsc.subcore_barrier()

    for b in range(NBUF - 1):
        pltpu.async_copy(htab.at[src_v.at[b]], rows_v.at[b], sem.at[b])

    def outer(i, carry):
        jj0 = i * NBUF
        for b in range(NBUF):
            jj = jj0 + b
            pltpu.make_async_copy(
                htab.at[src_v.at[jj]], rows_v.at[b], sem.at[b]).wait()
            nxt = jj + NBUF - 1
            nb = (b + NBUF - 1) % NBUF

            @pl.when(nxt < CH_PER_TILE)
            def _():
                pltpu.async_copy(
                    htab.at[src_v.at[nxt]], rows_v.at[nb], sem.at[nb])

            pltpu.sync_copy(rows_v.at[b], accum.at[dst_v.at[jj]], add=True)
        return carry

    lax.fori_loop(0, CH_PER_TILE // NBUF, outer, 0)
    plsc.subcore_barrier()
    pltpu.sync_copy(accum.at[pl.ds(s * ZROWS, ZROWS)],
                    out_hbm.at[c, pl.ds(s * ZROWS, ZROWS)])


@functools.cache
def _mp_call_builder():
    return functools.partial(
        pl.kernel,
        out_type=jax.ShapeDtypeStruct((NUM_SC, ACC_ROWS, HID), jnp.float32),
        mesh=plsc.VectorSubcoreMesh(core_axis_name="c", subcore_axis_name="s"),
        compiler_params=pltpu.CompilerParams(use_tc_tiling_on_sc=False),
        scratch_types=[
            pltpu.VMEM((CH_PER_TILE, CHUNK), jnp.int32),
            pltpu.VMEM((CH_PER_TILE, CHUNK), jnp.int32),
            pltpu.VMEM((NBUF, CHUNK, HID), jnp.float32),
            pltpu.VMEM_SHARED((N_NODES, HID), jnp.float32),
            pltpu.VMEM_SHARED((ACC_ROWS, HID), jnp.float32),
            pltpu.SemaphoreType.DMA((NBUF,)),
        ],
    )(_mp_body)


def _mp_call(h, src_p, dst_p, zblk):
    return _mp_call_builder()(h, src_p, dst_p, zblk)


def _mm1_body(x_ref, w_ref, o_ref):
    o_ref[:] = jnp.dot(x_ref[:], w_ref[:], preferred_element_type=jnp.float32)


def _mm2_body(p0_ref, p1_ref, b_ref, w_ref, o_ref):
    h = jnp.maximum(p0_ref[0] + p1_ref[0] + b_ref[:], 0.0)
    o_ref[:] = jnp.dot(h, w_ref[:], preferred_element_type=jnp.float32)


def _head_body(q0_ref, q1_ref, b2_ref, batch_ref, wf1_ref, bf1_ref,
               wf2_ref, bf2_ref, o_ref):
    h = jnp.maximum(q0_ref[0] + q1_ref[0] + b2_ref[:], 0.0)
    gid = lax.broadcasted_iota(jnp.int32, (N_NODES, N_GRAPHS), 1)
    oh = jnp.where(batch_ref[:] == gid, 1.0, 0.0)
    sums = lax.dot_general(oh, h, (((0,), (0,)), ((), ())),
                           preferred_element_type=jnp.float32)
    ones = jnp.ones((N_NODES, 8), jnp.float32)
    counts = lax.dot_general(oh, ones, (((0,), (0,)), ((), ())),
                             preferred_element_type=jnp.float32)[:, :1]
    pooled = sums / jnp.maximum(counts, 1.0)
    z = jnp.maximum(jnp.dot(pooled, wf1_ref[:],
                            preferred_element_type=jnp.float32) + bf1_ref[:], 0.0)
    logits = jnp.dot(z, wf2_ref[:],
                     preferred_element_type=jnp.float32) + bf2_ref[:]
    m = jnp.max(logits, axis=1, keepdims=True)
    lse = jnp.log(jnp.sum(jnp.exp(logits - m), axis=1, keepdims=True))
    o_ref[:] = logits - m - lse


_ROWB = 1000


def _mm1(x, W1):
    return pl.pallas_call(
        _mm1_body,
        grid=(N_NODES // _ROWB,),
        in_specs=[pl.BlockSpec((_ROWB, 128), lambda i: (i, 0)),
                  pl.BlockSpec((128, HID), lambda i: (0, 0))],
        out_specs=pl.BlockSpec((_ROWB, HID), lambda i: (i, 0)),
        out_shape=jax.ShapeDtypeStruct((N_NODES, HID), jnp.float32),
    )(x, W1)


def _mm2(p, b, W):
    return pl.pallas_call(
        _mm2_body,
        grid=(N_NODES // _ROWB,),
        in_specs=[pl.BlockSpec((1, _ROWB, HID), lambda i: (0, i, 0)),
                  pl.BlockSpec((1, _ROWB, HID), lambda i: (1, i, 0)),
                  pl.BlockSpec((1, HID), lambda i: (0, 0)),
                  pl.BlockSpec((HID, HID), lambda i: (0, 0))],
        out_specs=pl.BlockSpec((_ROWB, HID), lambda i: (i, 0)),
        out_shape=jax.ShapeDtypeStruct((N_NODES, HID), jnp.float32),
    )(p, p, b, W)


def _head(q, b2, batch2d, Wf1, bf1, Wf2, bf2):
    return pl.pallas_call(
        _head_body,
        grid=(1,),
        in_specs=[pl.BlockSpec((1, N_NODES, HID), lambda i: (0, 0, 0)),
                  pl.BlockSpec((1, N_NODES, HID), lambda i: (1, 0, 0)),
                  pl.BlockSpec((1, HID), lambda i: (0, 0)),
                  pl.BlockSpec((N_NODES, 1), lambda i: (0, 0)),
                  pl.BlockSpec((HID, 32), lambda i: (0, 0)),
                  pl.BlockSpec((1, 32), lambda i: (0, 0)),
                  pl.BlockSpec((32, N_CLASSES), lambda i: (0, 0)),
                  pl.BlockSpec((1, N_CLASSES), lambda i: (0, 0))],
        out_specs=pl.BlockSpec((N_GRAPHS, N_CLASSES), lambda i: (0, 0)),
        out_shape=jax.ShapeDtypeStruct((N_GRAPHS, N_CLASSES), jnp.float32),
    )(q, q, b2, batch2d, Wf1, bf1, Wf2, bf2)


def kernel(x, edge_index, batch, W1, b1, W2, b2, Wf1, bf1, Wf2, bf2):
    src = edge_index[0]
    dst = edge_index[1]
    pad = E_PAD - src.shape[0]
    src_p = jnp.concatenate(
        [src, jnp.zeros((pad,), jnp.int32)]).reshape(
        NUM_SC * NUM_TILES, CH_PER_TILE, CHUNK)
    dst_p = jnp.concatenate(
        [dst, jnp.full((pad,), N_NODES, jnp.int32)]).reshape(
        NUM_SC * NUM_TILES, CH_PER_TILE, CHUNK)
    zblk = jnp.zeros((ZROWS, HID), jnp.float32)

    b1r = b1.reshape(1, HID)
    b2r = b2.reshape(1, HID)
    bf1r = bf1.reshape(1, 32)
    bf2r = bf2.reshape(1, N_CLASSES)
    batch2d = batch.reshape(N_NODES, 1)

    h1 = _mm1(x, W1)
    p = _mp_call(h1, src_p, dst_p, zblk)
    h2 = _mm2(p, b1r, W2)
    q = _mp_call(h2, src_p, dst_p, zblk)
    return _head(q, b2r, batch2d, Wf1, bf1r, Wf2, bf2r)


# speedup vs baseline: 13.0986x; 1.0062x over previous
import functools

import jax
import jax.numpy as jnp
from jax import lax
from jax.experimental import pallas as pl
from jax.experimental.pallas import tpu as pltpu
from jax.experimental.pallas import tpu_sc as plsc

N_NODES = 10000
HID = 64
N_GRAPHS = 64
N_CLASSES = 10

NUM_SC = 2
NUM_TILES = 16
CHUNK = 128
CH_PER_TILE = 81
E_PER_TILE = CHUNK * CH_PER_TILE
E_PAD = NUM_SC * NUM_TILES * E_PER_TILE
NBUF = 3
HROWS = N_NODES // NUM_TILES
ACC_ROWS = 10048
ZROWS = ACC_ROWS // NUM_TILES


def _mp_body(h_hbm, src_hbm, dst_hbm, z_hbm, out_hbm,
             src_v, dst_v, rows_v, htab, accum, sem, ssem):
    c = lax.axis_index("c")
    s = lax.axis_index("s")
    g = c * NUM_TILES + s

    pltpu.sync_copy(z_hbm, accum.at[pl.ds(s * ZROWS, ZROWS)])
    pltpu.sync_copy(h_hbm.at[pl.ds(s * HROWS, HROWS)],
                    htab.at[pl.ds(s * HROWS, HROWS)])
    pltpu.sync_copy(src_hbm.at[g], src_v)
    pltpu.sync_copy(dst_hbm.at[g], dst_v)
    plsc.subcore_barrier()

    for b in range(NBUF - 1):
        pltpu.async_copy(htab.at[src_v.at[b]], rows_v.at[b], sem.at[b])

    def outer(i, carry):
        jj0 = i * NBUF
        for b in range(NBUF):
            jj = jj0 + b
            pltpu.make_async_copy(
                htab.at[src_v.at[jj]], rows_v.at[b], sem.at[b]).wait()
            nxt = jj + NBUF - 1
            nb = (b + NBUF - 1) % NBUF

            @pl.when(nxt < CH_PER_TILE)
            def _():
                @pl.when(jj >= 1)
                def _():
                    pltpu.make_async_copy(
                        rows_v.at[nb], accum.at[dst_v.at[nxt]],
                        ssem.at[nb]).wait()

                pltpu.async_copy(
                    htab.at[src_v.at[nxt]], rows_v.at[nb], sem.at[nb])

            pltpu.async_copy(rows_v.at[b], accum.at[dst_v.at[jj]],
                             ssem.at[b], add=True)
        return carry

    lax.fori_loop(0, CH_PER_TILE // NBUF, outer, 0)
    for b in range(NBUF):
        pltpu.make_async_copy(
            rows_v.at[b], accum.at[dst_v.at[b]], ssem.at[b]).wait()
    plsc.subcore_barrier()
    pltpu.sync_copy(accum.at[pl.ds(s * ZROWS, ZROWS)],
                    out_hbm.at[c, pl.ds(s * ZROWS, ZROWS)])


@functools.cache
def _mp_call_builder():
    return functools.partial(
        pl.kernel,
        out_type=jax.ShapeDtypeStruct((NUM_SC, ACC_ROWS, HID), jnp.float32),
        mesh=plsc.VectorSubcoreMesh(core_axis_name="c", subcore_axis_name="s"),
        compiler_params=pltpu.CompilerParams(use_tc_tiling_on_sc=False),
        scratch_types=[
            pltpu.VMEM((CH_PER_TILE, CHUNK), jnp.int32),
            pltpu.VMEM((CH_PER_TILE, CHUNK), jnp.int32),
            pltpu.VMEM((NBUF, CHUNK, HID), jnp.float32),
            pltpu.VMEM_SHARED((N_NODES, HID), jnp.float32),
            pltpu.VMEM_SHARED((ACC_ROWS, HID), jnp.float32),
            pltpu.SemaphoreType.DMA((NBUF,)),
            pltpu.SemaphoreType.DMA((NBUF,)),
        ],
    )(_mp_body)


def _mp_call(h, src_p, dst_p, zblk):
    return _mp_call_builder()(h, src_p, dst_p, zblk)


def _mm1_body(x_ref, w_ref, o_ref):
    o_ref[:] = jnp.dot(x_ref[:], w_ref[:], preferred_element_type=jnp.float32)


def _mm2_body(p0_ref, p1_ref, b_ref, w_ref, o_ref):
    h = jnp.maximum(p0_ref[0] + p1_ref[0] + b_ref[:], 0.0)
    o_ref[:] = jnp.dot(h, w_ref[:], preferred_element_type=jnp.float32)


def _head_body(q0_ref, q1_ref, b2_ref, batch_ref, wf1_ref, bf1_ref,
               wf2_ref, bf2_ref, o_ref):
    h = jnp.maximum(q0_ref[0] + q1_ref[0] + b2_ref[:], 0.0)
    gid = lax.broadcasted_iota(jnp.int32, (N_NODES, N_GRAPHS), 1)
    oh = jnp.where(batch_ref[:] == gid, 1.0, 0.0)
    sums = lax.dot_general(oh, h, (((0,), (0,)), ((), ())),
                           preferred_element_type=jnp.float32)
    ones = jnp.ones((N_NODES, 8), jnp.float32)
    counts = lax.dot_general(oh, ones, (((0,), (0,)), ((), ())),
                             preferred_element_type=jnp.float32)[:, :1]
    pooled = sums / jnp.maximum(counts, 1.0)
    z = jnp.maximum(jnp.dot(pooled, wf1_ref[:],
                            preferred_element_type=jnp.float32) + bf1_ref[:], 0.0)
    logits = jnp.dot(z, wf2_ref[:],
                     preferred_element_type=jnp.float32) + bf2_ref[:]
    m = jnp.max(logits, axis=1, keepdims=True)
    lse = jnp.log(jnp.sum(jnp.exp(logits - m), axis=1, keepdims=True))
    o_ref[:] = logits - m - lse


_ROWB = 1000


def _mm1(x, W1):
    return pl.pallas_call(
        _mm1_body,
        grid=(N_NODES // _ROWB,),
        in_specs=[pl.BlockSpec((_ROWB, 128), lambda i: (i, 0)),
                  pl.BlockSpec((128, HID), lambda i: (0, 0))],
        out_specs=pl.BlockSpec((_ROWB, HID), lambda i: (i, 0)),
        out_shape=jax.ShapeDtypeStruct((N_NODES, HID), jnp.float32),
    )(x, W1)


def _mm2(p, b, W):
    return pl.pallas_call(
        _mm2_body,
        grid=(N_NODES // _ROWB,),
        in_specs=[pl.BlockSpec((1, _ROWB, HID), lambda i: (0, i, 0)),
                  pl.BlockSpec((1, _ROWB, HID), lambda i: (1, i, 0)),
                  pl.BlockSpec((1, HID), lambda i: (0, 0)),
                  pl.BlockSpec((HID, HID), lambda i: (0, 0))],
        out_specs=pl.BlockSpec((_ROWB, HID), lambda i: (i, 0)),
        out_shape=jax.ShapeDtypeStruct((N_NODES, HID), jnp.float32),
    )(p, p, b, W)


def _head(q, b2, batch2d, Wf1, bf1, Wf2, bf2):
    return pl.pallas_call(
        _head_body,
        grid=(1,),
        in_specs=[pl.BlockSpec((1, N_NODES, HID), lambda i: (0, 0, 0)),
                  pl.BlockSpec((1, N_NODES, HID), lambda i: (1, 0, 0)),
                  pl.BlockSpec((1, HID), lambda i: (0, 0)),
                  pl.BlockSpec((N_NODES, 1), lambda i: (0, 0)),
                  pl.BlockSpec((HID, 32), lambda i: (0, 0)),
                  pl.BlockSpec((1, 32), lambda i: (0, 0)),
                  pl.BlockSpec((32, N_CLASSES), lambda i: (0, 0)),
                  pl.BlockSpec((1, N_CLASSES), lambda i: (0, 0))],
        out_specs=pl.BlockSpec((N_GRAPHS, N_CLASSES), lambda i: (0, 0)),
        out_shape=jax.ShapeDtypeStruct((N_GRAPHS, N_CLASSES), jnp.float32),
    )(q, q, b2, batch2d, Wf1, bf1, Wf2, bf2)


def kernel(x, edge_index, batch, W1, b1, W2, b2, Wf1, bf1, Wf2, bf2):
    src = edge_index[0]
    dst = edge_index[1]
    pad = E_PAD - src.shape[0]
    src_p = jnp.concatenate(
        [src, jnp.zeros((pad,), jnp.int32)]).reshape(
        NUM_SC * NUM_TILES, CH_PER_TILE, CHUNK)
    dst_p = jnp.concatenate(
        [dst, jnp.full((pad,), N_NODES, jnp.int32)]).reshape(
        NUM_SC * NUM_TILES, CH_PER_TILE, CHUNK)
    zblk = jnp.zeros((ZROWS, HID), jnp.float32)

    b1r = b1.reshape(1, HID)
    b2r = b2.reshape(1, HID)
    bf1r = bf1.reshape(1, 32)
    bf2r = bf2.reshape(1, N_CLASSES)
    batch2d = batch.reshape(N_NODES, 1)

    h1 = _mm1(x, W1)
    p = _mp_call(h1, src_p, dst_p, zblk)
    h2 = _mm2(p, b1r, W2)
    q = _mp_call(h2, src_p, dst_p, zblk)
    return _head(q, b2r, batch2d, Wf1, bf1r, Wf2, bf2r)


# speedup vs baseline: 13.7186x; 1.0473x over previous
import functools

import jax
import jax.numpy as jnp
from jax import lax
from jax.experimental import pallas as pl
from jax.experimental.pallas import tpu as pltpu
from jax.experimental.pallas import tpu_sc as plsc

N_NODES = 10000
HID = 64
N_GRAPHS = 64
N_CLASSES = 10

NUM_SC = 2
NUM_TILES = 16
CHUNK = 128
E_ROWS = 2500
CH_MAIN = 78
EXTRA_TILES = E_ROWS - CH_MAIN * NUM_SC * NUM_TILES
NBUF = 3
HROWS = N_NODES // NUM_TILES
ACC_ROWS = N_NODES
ZROWS = ACC_ROWS // NUM_TILES


def _mp_body(h_hbm, ei_hbm, z_hbm, out_hbm,
             src_v, dst_v, rows_v, htab, accum, sem, ssem):
    c = lax.axis_index("c")
    s = lax.axis_index("s")
    g = c * NUM_TILES + s

    pltpu.sync_copy(z_hbm, accum.at[pl.ds(s * ZROWS, ZROWS)])
    pltpu.sync_copy(h_hbm.at[pl.ds(s * HROWS, HROWS)],
                    htab.at[pl.ds(s * HROWS, HROWS)])
    pltpu.sync_copy(ei_hbm.at[0, pl.ds(g * CH_MAIN, CH_MAIN)],
                    src_v.at[pl.ds(0, CH_MAIN)])
    pltpu.sync_copy(ei_hbm.at[1, pl.ds(g * CH_MAIN, CH_MAIN)],
                    dst_v.at[pl.ds(0, CH_MAIN)])

    @pl.when(g < EXTRA_TILES)
    def _():
        pltpu.sync_copy(ei_hbm.at[0, CH_MAIN * NUM_SC * NUM_TILES + g],
                        src_v.at[CH_MAIN])
        pltpu.sync_copy(ei_hbm.at[1, CH_MAIN * NUM_SC * NUM_TILES + g],
                        dst_v.at[CH_MAIN])

    plsc.subcore_barrier()

    for b in range(NBUF - 1):
        pltpu.async_copy(htab.at[src_v.at[b]], rows_v.at[b], sem.at[b])

    def outer(i, carry):
        jj0 = i * NBUF
        for b in range(NBUF):
            jj = jj0 + b
            pltpu.make_async_copy(
                htab.at[src_v.at[jj]], rows_v.at[b], sem.at[b]).wait()
            nxt = jj + NBUF - 1
            nb = (b + NBUF - 1) % NBUF

            @pl.when(nxt < CH_MAIN)
            def _():
                @pl.when(jj >= 1)
                def _():
                    pltpu.make_async_copy(
                        rows_v.at[nb], accum.at[dst_v.at[nxt]],
                        ssem.at[nb]).wait()

                pltpu.async_copy(
                    htab.at[src_v.at[nxt]], rows_v.at[nb], sem.at[nb])

            pltpu.async_copy(rows_v.at[b], accum.at[dst_v.at[jj]],
                             ssem.at[b], add=True)
        return carry

    lax.fori_loop(0, CH_MAIN // NBUF, outer, 0)
    for b in range(NBUF):
        pltpu.make_async_copy(
            rows_v.at[b], accum.at[dst_v.at[b]], ssem.at[b]).wait()

    @pl.when(g < EXTRA_TILES)
    def _():
        pltpu.async_copy(htab.at[src_v.at[CH_MAIN]], rows_v.at[0],
                         sem.at[0])
        pltpu.make_async_copy(htab.at[src_v.at[CH_MAIN]], rows_v.at[0],
                              sem.at[0]).wait()
        pltpu.sync_copy(rows_v.at[0], accum.at[dst_v.at[CH_MAIN]], add=True)

    plsc.subcore_barrier()
    pltpu.sync_copy(accum.at[pl.ds(s * ZROWS, ZROWS)],
                    out_hbm.at[c, pl.ds(s * ZROWS, ZROWS)])


@functools.cache
def _mp_call_builder():
    return functools.partial(
        pl.kernel,
        out_type=jax.ShapeDtypeStruct((NUM_SC, ACC_ROWS, HID), jnp.float32),
        mesh=plsc.VectorSubcoreMesh(core_axis_name="c", subcore_axis_name="s"),
        compiler_params=pltpu.CompilerParams(use_tc_tiling_on_sc=False),
        scratch_types=[
            pltpu.VMEM((CH_MAIN + 1, CHUNK), jnp.int32),
            pltpu.VMEM((CH_MAIN + 1, CHUNK), jnp.int32),
            pltpu.VMEM((NBUF, CHUNK, HID), jnp.float32),
            pltpu.VMEM_SHARED((N_NODES, HID), jnp.float32),
            pltpu.VMEM_SHARED((ACC_ROWS, HID), jnp.float32),
            pltpu.SemaphoreType.DMA((NBUF,)),
            pltpu.SemaphoreType.DMA((NBUF,)),
        ],
    )(_mp_body)


def _mp_call(h, ei3, zblk):
    return _mp_call_builder()(h, ei3, zblk)


def _mm1_body(x_ref, w_ref, o_ref):
    o_ref[:] = jnp.dot(x_ref[:], w_ref[:], preferred_element_type=jnp.float32)


def _mm2_body(p0_ref, p1_ref, b_ref, w_ref, o_ref):
    h = jnp.maximum(p0_ref[0] + p1_ref[0] + b_ref[:], 0.0)
    o_ref[:] = jnp.dot(h, w_ref[:], preferred_element_type=jnp.float32)


def _head_body(q0_ref, q1_ref, b2_ref, batch_ref, wf1_ref, bf1_ref,
               wf2_ref, bf2_ref, o_ref):
    h = jnp.maximum(q0_ref[0] + q1_ref[0] + b2_ref[:], 0.0)
    gid = lax.broadcasted_iota(jnp.int32, (N_NODES, N_GRAPHS), 1)
    oh = jnp.where(batch_ref[:] == gid, 1.0, 0.0)
    sums = lax.dot_general(oh, h, (((0,), (0,)), ((), ())),
                           preferred_element_type=jnp.float32)
    ones = jnp.ones((N_NODES, 8), jnp.float32)
    counts = lax.dot_general(oh, ones, (((0,), (0,)), ((), ())),
                             preferred_element_type=jnp.float32)[:, :1]
    pooled = sums / jnp.maximum(counts, 1.0)
    z = jnp.maximum(jnp.dot(pooled, wf1_ref[:],
                            preferred_element_type=jnp.float32) + bf1_ref[:], 0.0)
    logits = jnp.dot(z, wf2_ref[:],
                     preferred_element_type=jnp.float32) + bf2_ref[:]
    m = jnp.max(logits, axis=1, keepdims=True)
    lse = jnp.log(jnp.sum(jnp.exp(logits - m), axis=1, keepdims=True))
    o_ref[:] = logits - m - lse


_ROWB = 1000


def _mm1(x, W1):
    return pl.pallas_call(
        _mm1_body,
        grid=(N_NODES // _ROWB,),
        in_specs=[pl.BlockSpec((_ROWB, 128), lambda i: (i, 0)),
                  pl.BlockSpec((128, HID), lambda i: (0, 0))],
        out_specs=pl.BlockSpec((_ROWB, HID), lambda i: (i, 0)),
        out_shape=jax.ShapeDtypeStruct((N_NODES, HID), jnp.float32),
    )(x, W1)


def _mm2(p, b, W):
    return pl.pallas_call(
        _mm2_body,
        grid=(N_NODES // _ROWB,),
        in_specs=[pl.BlockSpec((1, _ROWB, HID), lambda i: (0, i, 0)),
                  pl.BlockSpec((1, _ROWB, HID), lambda i: (1, i, 0)),
                  pl.BlockSpec((1, HID), lambda i: (0, 0)),
                  pl.BlockSpec((HID, HID), lambda i: (0, 0))],
        out_specs=pl.BlockSpec((_ROWB, HID), lambda i: (i, 0)),
        out_shape=jax.ShapeDtypeStruct((N_NODES, HID), jnp.float32),
    )(p, p, b, W)


def _head(q, b2, batch2d, Wf1, bf1, Wf2, bf2):
    return pl.pallas_call(
        _head_body,
        grid=(1,),
        in_specs=[pl.BlockSpec((1, N_NODES, HID), lambda i: (0, 0, 0)),
                  pl.BlockSpec((1, N_NODES, HID), lambda i: (1, 0, 0)),
                  pl.BlockSpec((1, HID), lambda i: (0, 0)),
                  pl.BlockSpec((N_NODES, 1), lambda i: (0, 0)),
                  pl.BlockSpec((HID, 32), lambda i: (0, 0)),
                  pl.BlockSpec((1, 32), lambda i: (0, 0)),
                  pl.BlockSpec((32, N_CLASSES), lambda i: (0, 0)),
                  pl.BlockSpec((1, N_CLASSES), lambda i: (0, 0))],
        out_specs=pl.BlockSpec((N_GRAPHS, N_CLASSES), lambda i: (0, 0)),
        out_shape=jax.ShapeDtypeStruct((N_GRAPHS, N_CLASSES), jnp.float32),
    )(q, q, b2, batch2d, Wf1, bf1, Wf2, bf2)


def kernel(x, edge_index, batch, W1, b1, W2, b2, Wf1, bf1, Wf2, bf2):
    ei3 = edge_index.reshape(2, E_ROWS, CHUNK)
    zblk = jnp.zeros((ZROWS, HID), jnp.float32)

    b1r = b1.reshape(1, HID)
    b2r = b2.reshape(1, HID)
    bf1r = bf1.reshape(1, 32)
    bf2r = bf2.reshape(1, N_CLASSES)
    batch2d = batch.reshape(N_NODES, 1)

    h1 = _mm1(x, W1)
    p = _mp_call(h1, ei3, zblk)
    h2 = _mm2(p, b1r, W2)
    q = _mp_call(h2, ei3, zblk)
    return _head(q, b2r, batch2d, Wf1, bf1r, Wf2, bf2r)


# speedup vs baseline: 14.4009x; 1.0497x over previous
import functools

import jax
import jax.numpy as jnp
from jax import lax
from jax.experimental import pallas as pl
from jax.experimental.pallas import tpu as pltpu
from jax.experimental.pallas import tpu_sc as plsc

N_NODES = 10000
HID = 64
N_GRAPHS = 64
N_CLASSES = 10

NUM_SC = 2
NUM_TILES = 16
CHUNK = 128
E_ROWS = 2500
CH_MAIN = 78
EXTRA_TILES = E_ROWS - CH_MAIN * NUM_SC * NUM_TILES
NBUF = 3
HROWS = N_NODES // NUM_TILES
ACC_ROWS = N_NODES
ZROWS = ACC_ROWS // NUM_TILES


def _mp_body(h_hbm, ei_hbm, z_hbm, out_hbm,
             src_v, dst_v, rows_v, htab, accum, sem, ssem):
    s = lax.axis_index("s")

    pltpu.sync_copy(z_hbm, accum.at[pl.ds(s * ZROWS, ZROWS)])
    pltpu.sync_copy(h_hbm.at[pl.ds(s * HROWS, HROWS)],
                    htab.at[pl.ds(s * HROWS, HROWS)])
    _mp_common(ei_hbm, out_hbm, src_v, dst_v, rows_v, htab, accum, sem, ssem)


def _mp_common(ei_hbm, out_hbm, src_v, dst_v, rows_v, htab, accum, sem, ssem):
    c = lax.axis_index("c")
    s = lax.axis_index("s")
    g = c * NUM_TILES + s
    pltpu.sync_copy(ei_hbm.at[0, pl.ds(g * CH_MAIN, CH_MAIN)],
                    src_v.at[pl.ds(0, CH_MAIN)])
    pltpu.sync_copy(ei_hbm.at[1, pl.ds(g * CH_MAIN, CH_MAIN)],
                    dst_v.at[pl.ds(0, CH_MAIN)])

    @pl.when(g < EXTRA_TILES)
    def _():
        pltpu.sync_copy(ei_hbm.at[0, CH_MAIN * NUM_SC * NUM_TILES + g],
                        src_v.at[CH_MAIN])
        pltpu.sync_copy(ei_hbm.at[1, CH_MAIN * NUM_SC * NUM_TILES + g],
                        dst_v.at[CH_MAIN])

    plsc.subcore_barrier()

    for b in range(NBUF - 1):
        pltpu.async_copy(htab.at[src_v.at[b]], rows_v.at[b], sem.at[b])

    def outer(i, carry):
        jj0 = i * NBUF
        for b in range(NBUF):
            jj = jj0 + b
            pltpu.make_async_copy(
                htab.at[src_v.at[jj]], rows_v.at[b], sem.at[b]).wait()
            nxt = jj + NBUF - 1
            nb = (b + NBUF - 1) % NBUF

            @pl.when(nxt < CH_MAIN)
            def _():
                @pl.when(jj >= 1)
                def _():
                    pltpu.make_async_copy(
                        rows_v.at[nb], accum.at[dst_v.at[nxt]],
                        ssem.at[nb]).wait()

                pltpu.async_copy(
                    htab.at[src_v.at[nxt]], rows_v.at[nb], sem.at[nb])

            pltpu.async_copy(rows_v.at[b], accum.at[dst_v.at[jj]],
                             ssem.at[b], add=True)
        return carry

    lax.fori_loop(0, CH_MAIN // NBUF, outer, 0)
    for b in range(NBUF):
        pltpu.make_async_copy(
            rows_v.at[b], accum.at[dst_v.at[b]], ssem.at[b]).wait()

    @pl.when(g < EXTRA_TILES)
    def _():
        pltpu.async_copy(htab.at[src_v.at[CH_MAIN]], rows_v.at[0],
                         sem.at[0])
        pltpu.make_async_copy(htab.at[src_v.at[CH_MAIN]], rows_v.at[0],
                              sem.at[0]).wait()
        pltpu.sync_copy(rows_v.at[0], accum.at[dst_v.at[CH_MAIN]], add=True)

    plsc.subcore_barrier()
    pltpu.sync_copy(accum.at[pl.ds(s * ZROWS, ZROWS)],
                    out_hbm.at[c, pl.ds(s * ZROWS, ZROWS)])


def _mp2_body(p_hbm, b1_hbm, ei_hbm, z_hbm, out_hbm,
              src_v, dst_v, rows_v, bias_v, htab, accum, sem, ssem):
    s = lax.axis_index("s")
    pltpu.sync_copy(z_hbm, accum.at[pl.ds(s * ZROWS, ZROWS)])
    pltpu.sync_copy(b1_hbm, bias_v)
    bias = [bias_v[0, pl.ds(16 * k, 16)] for k in range(HID // 16)]
    base = s * HROWS
    for t in range(5):
        r0 = base + t * 125
        pltpu.sync_copy(p_hbm.at[0, pl.ds(r0, 125)],
                        rows_v.at[0, pl.ds(0, 125)])
        pltpu.sync_copy(p_hbm.at[1, pl.ds(r0, 125)],
                        rows_v.at[1, pl.ds(0, 125)])

        def relu_row(i, carry):
            for k in range(HID // 16):
                col = pl.ds(16 * k, 16)
                rows_v[2, i, col] = jnp.maximum(
                    rows_v[0, i, col] + rows_v[1, i, col] + bias[k], 0.0)
            return carry

        lax.fori_loop(0, 125, relu_row, 0)
        pltpu.sync_copy(rows_v.at[2, pl.ds(0, 125)],
                        htab.at[pl.ds(r0, 125)])
    _mp_common(ei_hbm, out_hbm, src_v, dst_v, rows_v, htab, accum, sem, ssem)


@functools.cache
def _mp_call_builder():
    return functools.partial(
        pl.kernel,
        out_type=jax.ShapeDtypeStruct((NUM_SC, ACC_ROWS, HID), jnp.float32),
        mesh=plsc.VectorSubcoreMesh(core_axis_name="c", subcore_axis_name="s"),
        compiler_params=pltpu.CompilerParams(use_tc_tiling_on_sc=False),
        scratch_types=[
            pltpu.VMEM((CH_MAIN + 1, CHUNK), jnp.int32),
            pltpu.VMEM((CH_MAIN + 1, CHUNK), jnp.int32),
            pltpu.VMEM((NBUF, CHUNK, HID), jnp.float32),
            pltpu.VMEM_SHARED((N_NODES, HID), jnp.float32),
            pltpu.VMEM_SHARED((ACC_ROWS, HID), jnp.float32),
            pltpu.SemaphoreType.DMA((NBUF,)),
            pltpu.SemaphoreType.DMA((NBUF,)),
        ],
    )(_mp_body)


def _mp_call(h, ei3, zblk):
    return _mp_call_builder()(h, ei3, zblk)


@functools.cache
def _mp2_call_builder():
    return functools.partial(
        pl.kernel,
        out_type=jax.ShapeDtypeStruct((NUM_SC, ACC_ROWS, HID), jnp.float32),
        mesh=plsc.VectorSubcoreMesh(core_axis_name="c", subcore_axis_name="s"),
        compiler_params=pltpu.CompilerParams(use_tc_tiling_on_sc=False),
        scratch_types=[
            pltpu.VMEM((CH_MAIN + 1, CHUNK), jnp.int32),
            pltpu.VMEM((CH_MAIN + 1, CHUNK), jnp.int32),
            pltpu.VMEM((NBUF, CHUNK, HID), jnp.float32),
            pltpu.VMEM((1, HID), jnp.float32),
            pltpu.VMEM_SHARED((N_NODES, HID), jnp.float32),
            pltpu.VMEM_SHARED((ACC_ROWS, HID), jnp.float32),
            pltpu.SemaphoreType.DMA((NBUF,)),
            pltpu.SemaphoreType.DMA((NBUF,)),
        ],
    )(_mp2_body)


def _mp2_call(p, b1r, ei3, zblk):
    return _mp2_call_builder()(p, b1r, ei3, zblk)


def _mm1_body(x_ref, w_ref, o_ref):
    o_ref[:] = jnp.dot(x_ref[:], w_ref[:], preferred_element_type=jnp.float32)


def _head_body(q0_ref, q1_ref, w2_ref, b2_ref, batch_ref, wf1_ref, bf1_ref,
               wf2_ref, bf2_ref, o_ref):
    agg2 = jnp.dot(q0_ref[0] + q1_ref[0], w2_ref[:],
                   preferred_element_type=jnp.float32)
    h = jnp.maximum(agg2 + b2_ref[:], 0.0)
    gid = lax.broadcasted_iota(jnp.int32, (N_NODES, N_GRAPHS), 1)
    oh = jnp.where(batch_ref[:] == gid, 1.0, 0.0)
    sums = lax.dot_general(oh, h, (((0,), (0,)), ((), ())),
                           preferred_element_type=jnp.float32)
    ones = jnp.ones((N_NODES, 8), jnp.float32)
    counts = lax.dot_general(oh, ones, (((0,), (0,)), ((), ())),
                             preferred_element_type=jnp.float32)[:, :1]
    pooled = sums / jnp.maximum(counts, 1.0)
    z = jnp.maximum(jnp.dot(pooled, wf1_ref[:],
                            preferred_element_type=jnp.float32) + bf1_ref[:], 0.0)
    logits = jnp.dot(z, wf2_ref[:],
                     preferred_element_type=jnp.float32) + bf2_ref[:]
    m = jnp.max(logits, axis=1, keepdims=True)
    lse = jnp.log(jnp.sum(jnp.exp(logits - m), axis=1, keepdims=True))
    o_ref[:] = logits - m - lse


_ROWB = 1000


def _mm1(x, W1):
    return pl.pallas_call(
        _mm1_body,
        grid=(N_NODES // _ROWB,),
        in_specs=[pl.BlockSpec((_ROWB, 128), lambda i: (i, 0)),
                  pl.BlockSpec((128, HID), lambda i: (0, 0))],
        out_specs=pl.BlockSpec((_ROWB, HID), lambda i: (i, 0)),
        out_shape=jax.ShapeDtypeStruct((N_NODES, HID), jnp.float32),
    )(x, W1)


def _head(q, W2, b2, batch2d, Wf1, bf1, Wf2, bf2):
    return pl.pallas_call(
        _head_body,
        grid=(1,),
        in_specs=[pl.BlockSpec((1, N_NODES, HID), lambda i: (0, 0, 0)),
                  pl.BlockSpec((1, N_NODES, HID), lambda i: (1, 0, 0)),
                  pl.BlockSpec((HID, HID), lambda i: (0, 0)),
                  pl.BlockSpec((1, HID), lambda i: (0, 0)),
                  pl.BlockSpec((N_NODES, 1), lambda i: (0, 0)),
                  pl.BlockSpec((HID, 32), lambda i: (0, 0)),
                  pl.BlockSpec((1, 32), lambda i: (0, 0)),
                  pl.BlockSpec((32, N_CLASSES), lambda i: (0, 0)),
                  pl.BlockSpec((1, N_CLASSES), lambda i: (0, 0))],
        out_specs=pl.BlockSpec((N_GRAPHS, N_CLASSES), lambda i: (0, 0)),
        out_shape=jax.ShapeDtypeStruct((N_GRAPHS, N_CLASSES), jnp.float32),
    )(q, q, W2, b2, batch2d, Wf1, bf1, Wf2, bf2)


def kernel(x, edge_index, batch, W1, b1, W2, b2, Wf1, bf1, Wf2, bf2):
    ei3 = edge_index.reshape(2, E_ROWS, CHUNK)
    zblk = jnp.zeros((ZROWS, HID), jnp.float32)

    b1r = b1.reshape(1, HID)
    b2r = b2.reshape(1, HID)
    bf1r = bf1.reshape(1, 32)
    bf2r = bf2.reshape(1, N_CLASSES)
    batch2d = batch.reshape(N_NODES, 1)

    h1 = _mm1(x, W1)
    p = _mp_call(h1, ei3, zblk)
    q = _mp2_call(p, b1r, ei3, zblk)
    return _head(q, W2, b2r, batch2d, Wf1, bf1r, Wf2, bf2r)
